# baseline (speedup 1.0000x reference)
import functools

import jax
import jax.numpy as jnp
from jax import lax
from jax.experimental import pallas as pl
from jax.experimental.pallas import tpu as pltpu
from jax.experimental.pallas import tpu_sc as plsc

D = 64
CHB = 128
NG = 4
NS2 = 2


@functools.lru_cache(maxsize=None)
def _make(NW, NC, H, B):
    mesh = plsc.VectorSubcoreMesh(core_axis_name="c", subcore_axis_name="s")
    NT = H * (B // CHB)
    per_w = NT // NW
    assert NT % NW == 0 and per_w % NG == 0 and per_w >= 2 * NG

    @functools.partial(
        pl.kernel,
        mesh=mesh,
        compiler_params=pltpu.CompilerParams(
            use_tc_tiling_on_sc=True, needs_layout_passes=False),
        out_type=jax.ShapeDtypeStruct((H, D, B), jnp.float32),
        scratch_types=[
            pltpu.VMEM((per_w, CHB), jnp.int32),
            pltpu.VMEM((NG, CHB, 128), jnp.float32),
            pltpu.VMEM((NS2, D, CHB), jnp.float32),
            pltpu.SemaphoreType.DMA,
            pltpu.SemaphoreType.DMA,
        ],
    )
    def k(idx_hbm, table_hbm, out_hbm, idx_all, g_v, st_v, gsem, ssem):
        wid = lax.axis_index("s") * NC + lax.axis_index("c")
        t0 = wid * per_w
        pltpu.sync_copy(idx_hbm.at[pl.ds(t0, per_w)], idx_all)

        lanes = jax.lax.iota(jnp.int32, 16)
        rot = [((lanes + d) & 15) for d in range(16)]

        def fire_g(i, s):
            pltpu.async_copy(table_hbm.at[idx_all.at[i]], g_v.at[s], gsem)

        def wait_g(s):
            pltpu.make_async_copy(
                table_hbm.at[idx_all.at[0]], g_v.at[s], gsem).wait()

        def fire_s(i, s2):
            t = t0 + i
            h = t >> 7
            c = t & 127
            pltpu.async_copy(
                st_v.at[s2], out_hbm.at[h, :, pl.ds(c * CHB, CHB)], ssem)

        def wait_s(s2):
            pltpu.make_async_copy(
                st_v.at[s2], out_hbm.at[0, :, pl.ds(0, CHB)], ssem).wait()

        def transpose(s, s2):
            def bgroup(bg, cc):
                b_vec = lanes + 16 * bg
                for eg in range(D // 16):
                    e0 = eg * 16
                    e_vecs = [rot[d] + e0 for d in range(16)]
                    vals = [plsc.load_gather(g_v.at[s], [b_vec, ev])
                            for ev in e_vecs]
                    for d in range(16):
                        plsc.store_scatter(
                            st_v.at[s2], [e_vecs[d], b_vec], vals[d])
                return cc

            lax.fori_loop(0, CHB // 16, bgroup, 0)

        def step(i, s, s2, do_wait_store, do_fire_gather):
            wait_g(s)
            if do_wait_store:
                wait_s(s2)
            transpose(s, s2)
            fire_s(i, s2)
            if do_fire_gather:
                fire_g(i + NG, s)

        for b in range(NG):
            fire_g(b, b)
        for b in range(NG):
            step(b, b, b % NS2, b >= NS2, True)

        def group(j, carry):
            i0 = j * NG
            for b in range(NG):
                step(i0 + b, b, (i0 + b) % NS2, True, True)
            return carry

        lax.fori_loop(1, per_w // NG - 1, group, 0)

        i0 = per_w - NG
        for b in range(NG):
            step(i0 + b, b, (i0 + b) % NS2, True, False)

        for s2 in range(NS2):
            wait_s(s2)

    return k


def kernel(lookup, table):
    B, H = lookup.shape
    V, Dd = table.shape
    info = plsc.get_sparse_core_info()
    NC, NSc = info.num_cores, info.num_subcores
    NW = NC * NSc
    idx = lookup.T.reshape(H * (B // CHB), CHB)
    tableP = jnp.zeros((V, 128), jnp.float32).at[:, :Dd].set(table)
    out = _make(NW, NC, H, B)(idx, tableP)
    return out.transpose(2, 0, 1)


# speedup vs baseline: 1.4018x; 1.4018x over previous
import functools

import jax
import jax.numpy as jnp
from jax import lax
from jax.experimental import pallas as pl
from jax.experimental.pallas import tpu as pltpu
from jax.experimental.pallas import tpu_sc as plsc

D = 64
CHB = 128
NG = 4
NS2 = 4


@functools.lru_cache(maxsize=None)
def _make(NW, NC, H, B):
    mesh = plsc.VectorSubcoreMesh(core_axis_name="c", subcore_axis_name="s")
    NT = H * (B // CHB)
    per_w = NT // NW
    assert NT % NW == 0 and per_w % NG == 0 and per_w >= 2 * NG

    @functools.partial(
        pl.kernel,
        mesh=mesh,
        compiler_params=pltpu.CompilerParams(
            use_tc_tiling_on_sc=True, needs_layout_passes=False),
        out_type=jax.ShapeDtypeStruct((H, D, B), jnp.float32),
        scratch_types=[
            pltpu.VMEM((per_w, CHB), jnp.int32),
            pltpu.VMEM((NG, CHB, 128), jnp.float32),
            pltpu.VMEM((NS2, D, CHB), jnp.float32),
            pltpu.SemaphoreType.DMA,
            pltpu.SemaphoreType.DMA,
        ],
    )
    def k(idx_hbm, table_hbm, out_hbm, idx_all, g_v, st_v, gsem, ssem):
        wid = lax.axis_index("s") * NC + lax.axis_index("c")
        t0 = wid * per_w
        pltpu.sync_copy(idx_hbm.at[pl.ds(t0, per_w)], idx_all)

        lanes = jax.lax.iota(jnp.int32, 16)
        rot = [((lanes + d) & 15) for d in range(16)]

        def fire_g(i, s):
            pltpu.async_copy(table_hbm.at[idx_all.at[i]], g_v.at[s], gsem)

        def wait_g(s):
            pltpu.make_async_copy(
                table_hbm.at[idx_all.at[0]], g_v.at[s], gsem).wait()

        def fire_s(i, s2):
            t = t0 + i
            h = t >> 7
            c = t & 127
            pltpu.async_copy(
                st_v.at[s2], out_hbm.at[h, :, pl.ds(c * CHB, CHB)], ssem)

        def wait_s(s2):
            pltpu.make_async_copy(
                st_v.at[s2], out_hbm.at[0, :, pl.ds(0, CHB)], ssem).wait()

        def transpose(s, s2):
            def bgroup(bg, cc):
                b_vec = lanes + 16 * bg
                for eg in range(D // 16):
                    e0 = eg * 16
                    e_vecs = [rot[d] + e0 for d in range(16)]
                    vals = [plsc.load_gather(g_v.at[s], [b_vec, ev])
                            for ev in e_vecs]
                    for d in range(16):
                        plsc.store_scatter(
                            st_v.at[s2], [e_vecs[d], b_vec], vals[d])
                return cc

            lax.fori_loop(0, CHB // 16, bgroup, 0)

        def step(i, s, s2, do_wait_store, do_fire_gather):
            wait_g(s)
            if do_wait_store:
                wait_s(s2)
            transpose(s, s2)
            fire_s(i, s2)
            if do_fire_gather:
                fire_g(i + NG, s)

        for b in range(NG):
            fire_g(b, b)
        for b in range(NG):
            step(b, b, b % NS2, b >= NS2, True)

        def group(j, carry):
            i0 = j * NG
            for b in range(NG):
                step(i0 + b, b, (i0 + b) % NS2, True, True)
            return carry

        lax.fori_loop(1, per_w // NG - 1, group, 0)

        i0 = per_w - NG
        for b in range(NG):
            step(i0 + b, b, (i0 + b) % NS2, True, False)

        for s2 in range(NS2):
            wait_s(s2)

    return k


def kernel(lookup, table):
    B, H = lookup.shape
    V, Dd = table.shape
    info = plsc.get_sparse_core_info()
    NC, NSc = info.num_cores, info.num_subcores
    NW = NC * NSc
    idx = lookup.T.reshape(H * (B // CHB), CHB)
    tableP = jnp.pad(table, ((0, 0), (0, 128 - Dd)))
    out = _make(NW, NC, H, B)(idx, tableP)
    return out.transpose(2, 0, 1)
